# baseline (speedup 1.0000x reference)
import functools

import jax
import jax.numpy as jnp
from jax.experimental import pallas as pl
from jax.experimental.pallas import tpu as pltpu

_EMPH = 1.5
_SIGN = -2147483648

_KEY_LO = -1065353217
_KEY_HI = 1065353216


def _f32_key(e):
    bits = jax.lax.bitcast_convert_type(e, jnp.int32)
    return jnp.where(bits >= 0, bits,
                     jnp.bitwise_xor(jnp.bitwise_not(bits),
                                     jnp.int32(_SIGN)))


def _row_kernel(x_ref, w_ref, b_ref, o_ref, *, k, t, d):
    x = x_ref[0]
    w = w_ref[...]
    e_col = jnp.tanh(
        jax.lax.dot_general(x, w, (((1,), (0,)), ((), ())),
                            preferred_element_type=jnp.float32)
        + b_ref[0, 0])
    e = jnp.reshape(e_col, (32, t // 32))
    key = _f32_key(e)

    def body(_, lohi):
        lo, hi = lohi
        mid = lo + (hi - lo + 1) // 2
        cnt = jnp.sum((key >= mid).astype(jnp.int32))
        pred = cnt >= k
        return (jnp.where(pred, mid, lo), jnp.where(pred, hi, mid - 1))

    theta, _ = jax.lax.fori_loop(
        0, 31, body, (jnp.int32(_KEY_LO), jnp.int32(_KEY_HI)))

    p = jnp.exp(e)
    z = jnp.sum(p)
    u = jnp.where(key >= theta, p * _EMPH, p)
    u_col = jnp.reshape(u, (t, 1))
    s = jax.lax.dot_general(u_col, x, (((0,), (0,)), ((), ())),
                            preferred_element_type=jnp.float32)
    o_ref[0] = s * (1.0 / z)


def kernel(x, W, b):
    B, T, D = x.shape
    k = max(1, int(T * 0.1))
    b2 = jnp.reshape(b, (1, 1)).astype(jnp.float32)
    out = pl.pallas_call(
        functools.partial(_row_kernel, k=k, t=T, d=D),
        grid=(B,),
        in_specs=[
            pl.BlockSpec((1, T, D), lambda i: (i, 0, 0)),
            pl.BlockSpec((D, 1), lambda i: (0, 0)),
            pl.BlockSpec((1, 1), lambda i: (0, 0)),
        ],
        out_specs=pl.BlockSpec((1, 1, D), lambda i: (i, 0, 0)),
        out_shape=jax.ShapeDtypeStruct((B, 1, D), jnp.float32),
    )(x, W, b2)
    return out


# speedup vs baseline: 3.0999x; 3.0999x over previous
import functools

import jax
import jax.numpy as jnp
from jax.experimental import pallas as pl
from jax.experimental.pallas import tpu as pltpu

_EMPH = 1.5
_SIGN = -2147483648

_KEY_LO = -1065353217
_KEY_HI = 1065353216


def _f32_key(e):
    bits = jax.lax.bitcast_convert_type(e, jnp.int32)
    return jnp.where(bits >= 0, bits,
                     jnp.bitwise_xor(jnp.bitwise_not(bits),
                                     jnp.int32(_SIGN)))


def _row_kernel(x_ref, w_ref, b_ref, o_ref, xw_scr, u_scr, *, k, t, d):
    x = x_ref[0]
    w = w_ref[...]
    xw_col = jax.lax.dot_general(x, w, (((1,), (0,)), ((), ())),
                                 preferred_element_type=jnp.float32)
    xw_scr[...] = jnp.reshape(xw_col, (32, t // 32))
    xw = xw_scr[...]
    e = jnp.tanh(xw + b_ref[0, 0])
    key = _f32_key(e)

    lo = jnp.full((1, 1), _KEY_LO, jnp.int32)
    hi = jnp.full((1, 1), _KEY_HI, jnp.int32)
    for _ in range(31):
        mid = lo + (hi - lo + 1) // 2
        cnt = jnp.sum((key >= mid).astype(jnp.int32), keepdims=True,
                      axis=(0, 1))
        pred = cnt >= k
        lo = jnp.where(pred, mid, lo)
        hi = jnp.where(pred, hi, mid - 1)
    theta = lo

    p = jnp.exp(e)
    z = jnp.sum(p, keepdims=True, axis=(0, 1))
    u = jnp.where(key >= theta, p * _EMPH, p)
    u_scr[...] = jnp.reshape(u, (1, t))
    s = jax.lax.dot_general(u_scr[...], x, (((1,), (0,)), ((), ())),
                            preferred_element_type=jnp.float32)
    o_ref[0] = s * (1.0 / z)


def kernel(x, W, b):
    B, T, D = x.shape
    k = max(1, int(T * 0.1))
    b2 = jnp.reshape(b, (1, 1)).astype(jnp.float32)
    out = pl.pallas_call(
        functools.partial(_row_kernel, k=k, t=T, d=D),
        grid=(B,),
        in_specs=[
            pl.BlockSpec((1, T, D), lambda i: (i, 0, 0)),
            pl.BlockSpec((D, 1), lambda i: (0, 0)),
            pl.BlockSpec((1, 1), lambda i: (0, 0)),
        ],
        out_specs=pl.BlockSpec((1, 1, D), lambda i: (i, 0, 0)),
        out_shape=jax.ShapeDtypeStruct((B, 1, D), jnp.float32),
        scratch_shapes=[
            pltpu.VMEM((32, T // 32), jnp.float32),
            pltpu.VMEM((1, T), jnp.float32),
        ],
    )(x, W, b2)
    return out


# speedup vs baseline: 4.1945x; 1.3531x over previous
import functools

import jax
import jax.numpy as jnp
from jax.experimental import pallas as pl
from jax.experimental.pallas import tpu as pltpu

_EMPH = 1.5
_SIGN = -2147483648

_KEY_LO = -1065353217
_KEY_HI = 1065353216


def _f32_key(e):
    bits = jax.lax.bitcast_convert_type(e, jnp.int32)
    return jnp.where(bits >= 0, bits,
                     jnp.bitwise_xor(jnp.bitwise_not(bits),
                                     jnp.int32(_SIGN)))


def _row_kernel(x_ref, w_ref, b_ref, o_ref, xw_scr, u_scr, *, k, t, d):
    x = x_ref[0]
    w = w_ref[...]
    xw_col = jax.lax.dot_general(x, w, (((1,), (0,)), ((), ())),
                                 preferred_element_type=jnp.float32)
    xw_scr[...] = jnp.reshape(xw_col, (32, t // 32))
    xw = xw_scr[...]
    e = jnp.tanh(xw + b_ref[0, 0])
    key = _f32_key(e)

    lo = jnp.full((1, 1), _KEY_LO, jnp.int32)
    hi = jnp.full((1, 1), _KEY_HI, jnp.int32)
    for _ in range(13):
        width = hi - lo + 1
        step = jnp.maximum(width >> 3, 1)
        ms = [lo + step * j for j in range(1, 8)]
        preds = [
            jnp.sum((key >= m).astype(jnp.int32), keepdims=True,
                    axis=(0, 1)) >= k
            for m in ms
        ]
        new_lo = lo
        for m, pred in zip(ms, preds):
            new_lo = jnp.where(pred, m, new_lo)
        new_hi = ms[0] - 1
        for m_next, pred in zip(ms[1:], preds[:-1]):
            new_hi = jnp.where(pred, m_next - 1, new_hi)
        new_hi = jnp.where(preds[-1], hi, new_hi)
        lo, hi = new_lo, new_hi
    theta = lo

    p = jnp.exp(e)
    z = jnp.sum(p, keepdims=True, axis=(0, 1))
    u = jnp.where(key >= theta, p * _EMPH, p)
    u_scr[...] = jnp.reshape(u, (1, t))
    s = jax.lax.dot_general(u_scr[...], x, (((1,), (0,)), ((), ())),
                            preferred_element_type=jnp.float32)
    o_ref[0] = s * (1.0 / z)


def kernel(x, W, b):
    B, T, D = x.shape
    k = max(1, int(T * 0.1))
    b2 = jnp.reshape(b, (1, 1)).astype(jnp.float32)
    out = pl.pallas_call(
        functools.partial(_row_kernel, k=k, t=T, d=D),
        grid=(B,),
        in_specs=[
            pl.BlockSpec((1, T, D), lambda i: (i, 0, 0)),
            pl.BlockSpec((D, 1), lambda i: (0, 0)),
            pl.BlockSpec((1, 1), lambda i: (0, 0)),
        ],
        out_specs=pl.BlockSpec((1, 1, D), lambda i: (i, 0, 0)),
        out_shape=jax.ShapeDtypeStruct((B, 1, D), jnp.float32),
        scratch_shapes=[
            pltpu.VMEM((32, T // 32), jnp.float32),
            pltpu.VMEM((1, T), jnp.float32),
        ],
    )(x, W, b2)
    return out


# speedup vs baseline: 5.8600x; 1.3971x over previous
import functools

import jax
import jax.numpy as jnp
from jax.experimental import pallas as pl
from jax.experimental.pallas import tpu as pltpu

_EMPH = 1.5
_SIGN = -2147483648

_KEY_LO = -1065353217
_KEY_HI = 1065353216

_CHUNKS = 8
_ROUNDS = 9


def _f32_key(e):
    bits = jax.lax.bitcast_convert_type(e, jnp.int32)
    return jnp.where(bits >= 0, bits,
                     jnp.bitwise_xor(jnp.bitwise_not(bits),
                                     jnp.int32(_SIGN)))


def _search_init():
    return (jnp.full((1, 1), _KEY_LO, jnp.int32),
            jnp.full((1, 1), _KEY_HI, jnp.int32))


def _search_round(key, k, state):
    lo, hi = state
    width = hi - lo + 1
    step = jnp.maximum(width >> 4, 1)
    ms = [lo + step * j for j in range(1, 16)]
    preds = [
        jnp.sum((key >= m).astype(jnp.int32), keepdims=True,
                axis=(0, 1)) >= k
        for m in ms
    ]
    new_lo = lo
    for m, pred in zip(ms, preds):
        new_lo = jnp.where(pred, m, new_lo)
    new_hi = ms[0] - 1
    for m_next, pred in zip(ms[1:], preds[:-1]):
        new_hi = jnp.where(pred, m_next - 1, new_hi)
    new_hi = jnp.where(preds[-1], hi, new_hi)
    return new_lo, new_hi


def _mv_chunk(x_ref, row, w, xw_scr, c, ck):
    xc = x_ref[row, c * ck:(c + 1) * ck, :]
    mv = jax.lax.dot_general(xc, w, (((1,), (0,)), ((), ())),
                             preferred_element_type=jnp.float32)
    xw_scr[c * (ck // 128):(c + 1) * (ck // 128), :] = (
        jnp.reshape(mv, (ck // 128, 128)))


def _dot2_chunk(x_ref, row, u_scr, c, ck, acc):
    uc = u_scr[:, c * ck:(c + 1) * ck]
    xc = x_ref[row, c * ck:(c + 1) * ck, :]
    part = jax.lax.dot_general(uc, xc, (((1,), (0,)), ((), ())),
                               preferred_element_type=jnp.float32)
    return part if acc is None else acc + part


def _finish_e(xw_scr, bias):
    e = jnp.tanh(xw_scr[...] + bias)
    return e, _f32_key(e)


def _weights(e, key, theta):
    p = jnp.exp(e)
    z = jnp.sum(p, keepdims=True, axis=(0, 1))
    u = jnp.where(key >= theta, p * _EMPH, p)
    return u, z


def _pair_kernel(x_ref, w_ref, b_ref, o_ref,
                 xw_scr0, u_scr0, xw_scr1, u_scr1, *, k, t, d):
    w = w_ref[...]
    bias = b_ref[0, 0]
    ck = t // _CHUNKS

    for c in range(_CHUNKS):
        _mv_chunk(x_ref, 0, w, xw_scr0, c, ck)
    e0, key0 = _finish_e(xw_scr0, bias)

    st0 = _search_init()
    for c in range(_CHUNKS):
        _mv_chunk(x_ref, 1, w, xw_scr1, c, ck)
        st0 = _search_round(key0, k, st0)
    for _ in range(_ROUNDS - _CHUNKS):
        st0 = _search_round(key0, k, st0)
    theta0 = st0[0]
    e1, key1 = _finish_e(xw_scr1, bias)

    u0, z0 = _weights(e0, key0, theta0)
    u_scr0[...] = jnp.reshape(u0, (1, t))
    st1 = _search_init()
    s0 = None
    for c in range(_CHUNKS):
        s0 = _dot2_chunk(x_ref, 0, u_scr0, c, ck, s0)
        st1 = _search_round(key1, k, st1)
    for _ in range(_ROUNDS - _CHUNKS):
        st1 = _search_round(key1, k, st1)
    theta1 = st1[0]
    o_ref[0] = s0 * (1.0 / z0)

    u1, z1 = _weights(e1, key1, theta1)
    u_scr1[...] = jnp.reshape(u1, (1, t))
    s1 = None
    for c in range(_CHUNKS):
        s1 = _dot2_chunk(x_ref, 1, u_scr1, c, ck, s1)
    o_ref[1] = s1 * (1.0 / z1)


def kernel(x, W, b):
    B, T, D = x.shape
    k = max(1, int(T * 0.1))
    b2 = jnp.reshape(b, (1, 1)).astype(jnp.float32)
    out = pl.pallas_call(
        functools.partial(_pair_kernel, k=k, t=T, d=D),
        grid=(B // 2,),
        in_specs=[
            pl.BlockSpec((2, T, D), lambda i: (i, 0, 0)),
            pl.BlockSpec((D, 1), lambda i: (0, 0)),
            pl.BlockSpec((1, 1), lambda i: (0, 0)),
        ],
        out_specs=pl.BlockSpec((2, 1, D), lambda i: (i, 0, 0)),
        out_shape=jax.ShapeDtypeStruct((B, 1, D), jnp.float32),
        scratch_shapes=[
            pltpu.VMEM((T // 128, 128), jnp.float32),
            pltpu.VMEM((1, T), jnp.float32),
            pltpu.VMEM((T // 128, 128), jnp.float32),
            pltpu.VMEM((1, T), jnp.float32),
        ],
    )(x, W, b2)
    return out
